# baseline (speedup 1.0000x reference)
import functools

import jax
import jax.numpy as jnp
from jax.experimental import pallas as pl
from jax.experimental.pallas import tpu as pltpu
from jax.experimental.pallas import tpu_sc as plsc

N_PRIORS_C = 21420
P_PAD = 21504
BATCH_C = 8
N_OBJ_C = 16
N_IGN_C = 4
N_CLASSES_C = 4
THRESHOLD_C = 0.4
NEG_POS_RATIO_C = 2
THETA_C = 0.1
ATT_HW = 56 * 96
NCHUNK = P_PAD // 16


def _pairwise_iou(bx1, by1, bx2, by2, px1, py1, px2, py2):
    lt_x = jnp.maximum(bx1, px1)
    lt_y = jnp.maximum(by1, py1)
    rb_x = jnp.minimum(bx2, px2)
    rb_y = jnp.minimum(by2, py2)
    inter = jnp.clip(rb_x - lt_x, 0.0, None) * jnp.clip(rb_y - lt_y, 0.0, None)
    area_b = (bx2 - bx1) * (by2 - by1)
    area_p = (px2 - px1) * (py2 - py1)
    union = area_b + area_p - inter
    return inter / union


def _image_kernel(locs_ref, scores_ref, att_ref, boxes_ref, labels_ref,
                  ign_ref, priors_ref, cn_ref, npos_ref, tot_ref, acc_ref):
    i = pl.program_id(0)

    @pl.when(i == 0)
    def _init():
        acc_ref[0] = 0.0
        acc_ref[1] = 0.0
        acc_ref[2] = 0.0
        acc_ref[3] = 0.0

    lane = jax.lax.broadcasted_iota(jnp.int32, (1, P_PAD), 1)
    lane_valid = lane < N_PRIORS_C

    pcx = priors_ref[0:1, :]
    pcy = priors_ref[1:2, :]
    pw = priors_ref[2:3, :]
    ph = priors_ref[3:4, :]
    px1 = pcx - pw * 0.5
    py1 = pcy - ph * 0.5
    px2 = pcx + pw * 0.5
    py2 = pcy + ph * 0.5

    b = boxes_ref[0]
    bx1 = b[:, 0:1]
    by1 = b[:, 1:2]
    bx2 = b[:, 2:3]
    by2 = b[:, 3:4]

    ov = _pairwise_iou(bx1, by1, bx2, by2, px1, py1, px2, py2)

    iota_obj = jax.lax.broadcasted_iota(jnp.int32, (N_OBJ_C, P_PAD), 0)
    iota_pri = jax.lax.broadcasted_iota(jnp.int32, (N_OBJ_C, P_PAD), 1)

    ofp = jnp.max(ov, axis=0, keepdims=True)
    obj_fp = jnp.min(jnp.where(ov == ofp, iota_obj, N_OBJ_C), axis=0,
                     keepdims=True)

    ofo = jnp.max(ov, axis=1, keepdims=True)
    pfo = jnp.min(jnp.where(ov == ofo, iota_pri, P_PAD), axis=1,
                  keepdims=True)
    valid = ofo > 0.0

    c = valid.astype(jnp.int32)
    for s in (1, 2, 4, 8):
        shifted = jnp.concatenate(
            [jnp.zeros((s, 1), jnp.int32), c[: N_OBJ_C - s, :]], axis=0)
        c = c + shifted
    rank = c - 1

    obj_j = jax.lax.broadcasted_iota(jnp.int32, (N_OBJ_C, 1), 0)
    match = pfo == lane
    j_sel = jnp.max(jnp.where(match, obj_j, -1), axis=0, keepdims=True)
    onehot2 = (j_sel == iota_obj).astype(jnp.float32)
    w2 = jnp.concatenate([valid.astype(jnp.float32),
                          rank.astype(jnp.float32)], axis=1)
    g2 = jax.lax.dot_general(w2, onehot2, (((0,), (0,)), ((), ())),
                             preferred_element_type=jnp.float32)
    valid_sel = g2[0:1, :] >= 0.5
    ofp = jnp.where(valid_sel, 1.0, ofp)
    obj_f = jnp.where(valid_sel, g2[1:2, :], obj_fp.astype(jnp.float32))

    iota_obj_f = iota_obj.astype(jnp.float32)
    onehot = (obj_f == iota_obj_f).astype(jnp.float32)
    labels_col = labels_ref[0].astype(jnp.float32)
    w5 = jnp.concatenate([labels_col, bx1, by1, bx2, by2], axis=1)
    g5 = jax.lax.dot_general(w5, onehot, (((0,), (0,)), ((), ())),
                             preferred_element_type=jnp.float32)
    lab = jnp.where(ofp < THRESHOLD_C, 0.0, g5[0:1, :])
    tx1 = g5[1:2, :]
    ty1 = g5[2:3, :]
    tx2 = g5[3:4, :]
    ty2 = g5[4:5, :]

    pos = lab > 0.0
    posf = pos.astype(jnp.float32)
    n_pos = jnp.sum(posf)

    g = ign_ref[0]
    gx1 = g[:, 0:1]
    gy1 = g[:, 1:2]
    gx2 = g[:, 2:3]
    gy2 = g[:, 3:4]
    i_x = jnp.clip(jnp.minimum(gx2, px2) - jnp.maximum(gx1, px1), 0.0, None)
    i_y = jnp.clip(jnp.minimum(gy2, py2) - jnp.maximum(gy1, py1), 0.0, None)
    inter_g = i_x * i_y
    area_sum = (gx2 - gx1) * (gy2 - gy1) + (px2 - px1) * (py2 - py1)
    ign = jnp.max(jnp.where(11.0 * inter_g >= area_sum, 1, 0), axis=0,
                  keepdims=True) > 0

    gl = locs_ref[0]
    d_cx = gl[0:1, :] * pw / 10.0 + pcx
    d_cy = gl[1:2, :] * ph / 10.0 + pcy
    d_w = jnp.exp(gl[2:3, :] / 5.0) * pw
    d_h = jnp.exp(gl[3:4, :] / 5.0) * ph
    dx1 = d_cx - d_w * 0.5
    dy1 = d_cy - d_h * 0.5
    dx2 = d_cx + d_w * 0.5
    dy2 = d_cy + d_h * 0.5

    ix1 = jnp.maximum(dx1, tx1)
    iy1 = jnp.maximum(dy1, ty1)
    ix2 = jnp.minimum(dx2, tx2)
    iy2 = jnp.minimum(dy2, ty2)
    inter = jnp.clip(ix2 - ix1, 0.0, None) * jnp.clip(iy2 - iy1, 0.0, None)
    ap = (dx2 - dx1) * (dy2 - dy1)
    at = (tx2 - tx1) * (ty2 - ty1)
    union = ap + at - inter
    iou = inter / (union + 1e-9)
    cxp = (dx1 + dx2) * 0.5
    cyp = (dy1 + dy2) * 0.5
    cxt = (tx1 + tx2) * 0.5
    cyt = (ty1 + ty2) * 0.5
    d2 = (cxp - cxt) ** 2 + (cyp - cyt) ** 2
    ex1 = jnp.minimum(dx1, tx1)
    ey1 = jnp.minimum(dy1, ty1)
    ex2 = jnp.maximum(dx2, tx2)
    ey2 = jnp.maximum(dy2, ty2)
    c2 = (ex2 - ex1) ** 2 + (ey2 - ey1) ** 2 + 1e-7
    diou = 1.0 - iou + d2 / c2
    loc_sum = jnp.sum(diou * posf)

    s = scores_ref[0]
    s0 = s[0:1, :]
    s1 = s[1:2, :]
    s2 = s[2:3, :]
    s3 = s[3:4, :]
    m = jnp.maximum(jnp.maximum(s0, s1), jnp.maximum(s2, s3))
    lse = m + jnp.log(jnp.exp(s0 - m) + jnp.exp(s1 - m)
                      + jnp.exp(s2 - m) + jnp.exp(s3 - m))
    picked = jnp.where(lab == 0.0, s0, 0.0) + jnp.where(lab == 1.0, s1, 0.0) \
        + jnp.where(lab == 2.0, s2, 0.0) + jnp.where(lab == 3.0, s3, 0.0)
    ce = lse - picked
    conf_pos_sum = jnp.sum(ce * posf)

    neg_mask = jnp.logical_not(pos | ign) & lane_valid
    cn_ref[0] = jnp.where(neg_mask, ce, 0.0)
    npos_ref[0] = jnp.full((1, 128), n_pos, jnp.float32)

    a = att_ref[0]
    seg = -jnp.sum(jnp.clip(jnp.log(1.0 - a), -100.0, None))

    acc_ref[0] = acc_ref[0] + n_pos
    acc_ref[1] = acc_ref[1] + loc_sum
    acc_ref[2] = acc_ref[2] + conf_pos_sum
    acc_ref[3] = acc_ref[3] + seg

    @pl.when(i == BATCH_C - 1)
    def _fin():
        tot_ref[0, 0] = acc_ref[0]
        tot_ref[0, 1] = acc_ref[1]
        tot_ref[0, 2] = acc_ref[2]
        tot_ref[0, 3] = acc_ref[3]


def _sc_topk_kernel(cn_hbm, npos_hbm, out_hbm, vals, npv, hcnt, hsum, orow,
                    sem):
    wid = jax.lax.axis_index("s") * 2 + jax.lax.axis_index("c")

    @pl.when(wid < BATCH_C)
    def _active():
        img = wid
        pltpu.sync_copy(cn_hbm.at[img], vals)
        pltpu.sync_copy(npos_hbm.at[img], npv)

        lanes = jax.lax.broadcasted_iota(jnp.int32, (16,), 0)
        ones_i = jnp.ones((16,), jnp.int32)
        zeros_i = jnp.zeros((16,), jnp.int32)
        zeros_f = jnp.zeros((16,), jnp.float32)
        lane_base = lanes * 256

        n_pos = jnp.sum(jnp.where(lanes == 0, npv[...], 0.0))
        k = jnp.minimum(
            NEG_POS_RATIO_C * n_pos.astype(jnp.int32), N_PRIORS_C)

        def _zero_hists():
            def zbody(bb, carry):
                hcnt[pl.ds(bb * 16, 16)] = zeros_i
                hsum[pl.ds(bb * 16, 16)] = zeros_f
                return carry
            jax.lax.fori_loop(0, 256, zbody, 0)

        def _suffix_stats(kk):
            ccnt = []
            csum = []
            for j in range(16):
                ac = zeros_i
                asm = zeros_f
                for l in range(16):
                    ac = ac + hcnt[pl.ds(l * 256 + j * 16, 16)]
                    asm = asm + hsum[pl.ds(l * 256 + j * 16, 16)]
                ccnt.append(ac)
                csum.append(asm)
            carry = jnp.int32(0)
            nb = jnp.int32(0)
            sgrp = [None] * 16
            for j in range(15, -1, -1):
                r = jax.lax.rev(ccnt[j], (0,))
                sfx = jax.lax.rev(jnp.cumsum(r), (0,)) + carry
                sgrp[j] = sfx
                carry = carry + jnp.sum(ccnt[j])
            for j in range(16):
                nb = nb + jnp.sum((sgrp[j] >= kk).astype(jnp.int32))
            bstar = nb - 1
            cnt_above = jnp.int32(0)
            sum_above = jnp.float32(0.0)
            bcnt = jnp.int32(0)
            bsum = jnp.float32(0.0)
            for j in range(16):
                binv = lanes + j * 16
                above = binv > bstar
                at_b = binv == bstar
                cnt_above = cnt_above + jnp.sum(
                    jnp.where(above, ccnt[j], 0))
                sum_above = sum_above + jnp.sum(
                    jnp.where(above, csum[j], 0.0))
                bcnt = bcnt + jnp.sum(jnp.where(at_b, ccnt[j], 0))
                bsum = bsum + jnp.sum(jnp.where(at_b, csum[j], 0.0))
            return bstar, cnt_above, sum_above, bcnt, bsum

        _zero_hists()

        def pass1(cc, carry):
            v = vals[pl.ds(cc * 16, 16)]
            bits = plsc.bitcast(v, jnp.int32)
            idx = jax.lax.shift_right_logical(bits, 23) + lane_base
            plsc.addupdate_scatter(hcnt, [idx], ones_i)
            plsc.addupdate_scatter(hsum, [idx], v)
            return carry
        jax.lax.fori_loop(0, NCHUNK, pass1, 0)

        bstar, cnt_ab1, sum_ab1, _, _ = _suffix_stats(k)

        _zero_hists()
        k2 = k - cnt_ab1

        def pass2(cc, carry):
            v = vals[pl.ds(cc * 16, 16)]
            bits = plsc.bitcast(v, jnp.int32)
            coarse = jax.lax.shift_right_logical(bits, 23)
            fine = jax.lax.shift_right_logical(bits, 15) & 0xFF
            msk = coarse == bstar
            idx = fine + lane_base
            plsc.addupdate_scatter(hcnt, [idx], ones_i, mask=msk)
            plsc.addupdate_scatter(hsum, [idx], v, mask=msk)
            return carry
        jax.lax.fori_loop(0, NCHUNK, pass2, 0)

        _, cnt_ab2, sum_ab2, bcnt, bsum = _suffix_stats(k2)

        ties_v = jnp.full((16,), (k2 - cnt_ab2).astype(jnp.float32))
        bsum_v = jnp.full((16,), bsum)
        bcnt_v = jnp.maximum(jnp.full((16,), bcnt.astype(jnp.float32)), 1.0)
        t_v = bsum_v / bcnt_v
        sum_v = jnp.full((16,), sum_ab1 + sum_ab2)
        k_v = jnp.full((16,), k)
        topk_v = jnp.where(k_v > 0, sum_v + ties_v * t_v,
                           jnp.zeros((16,), jnp.float32))
        orow[...] = topk_v
        pltpu.sync_copy(orow, out_hbm.at[img])


_sc_topk = functools.partial(
    pl.kernel,
    mesh=plsc.VectorSubcoreMesh(core_axis_name="c", subcore_axis_name="s"),
    out_type=jax.ShapeDtypeStruct((BATCH_C, 16), jnp.float32),
    scratch_types=[
        pltpu.VMEM((P_PAD,), jnp.float32),
        pltpu.VMEM((16,), jnp.float32),
        pltpu.VMEM((4096,), jnp.int32),
        pltpu.VMEM((4096,), jnp.float32),
        pltpu.VMEM((16,), jnp.float32),
        pltpu.SemaphoreType.DMA,
    ],
    compiler_params=pltpu.CompilerParams(needs_layout_passes=False),
)(_sc_topk_kernel)


@jax.jit
def kernel(odm_locs, odm_scores, attention_map, boxes, labels,
           ignored_regions, priors_cxcy):
    B, P, _ = odm_scores.shape
    pad = P_PAD - P

    locs_t = jnp.pad(jnp.transpose(odm_locs, (0, 2, 1)),
                     ((0, 0), (0, 0), (0, pad)))
    scores_t = jnp.pad(jnp.transpose(odm_scores, (0, 2, 1)),
                       ((0, 0), (0, 0), (0, pad)))
    sentinel = jnp.tile(jnp.array([[-9.0], [-9.0], [1.0], [1.0]],
                                  jnp.float32), (1, pad))
    priors_t = jnp.concatenate(
        [jnp.transpose(priors_cxcy, (1, 0)), sentinel], axis=1)
    att = attention_map.reshape(B, 1, ATT_HW)
    labels_c = labels.astype(jnp.int32).reshape(B, N_OBJ_C, 1)

    cn, npos, tot = pl.pallas_call(
        _image_kernel,
        grid=(B,),
        in_specs=[
            pl.BlockSpec((1, 4, P_PAD), lambda i: (i, 0, 0)),
            pl.BlockSpec((1, 4, P_PAD), lambda i: (i, 0, 0)),
            pl.BlockSpec((1, 1, ATT_HW), lambda i: (i, 0, 0)),
            pl.BlockSpec((1, N_OBJ_C, 4), lambda i: (i, 0, 0)),
            pl.BlockSpec((1, N_OBJ_C, 1), lambda i: (i, 0, 0)),
            pl.BlockSpec((1, N_IGN_C, 4), lambda i: (i, 0, 0)),
            pl.BlockSpec((4, P_PAD), lambda i: (0, 0)),
        ],
        out_specs=[
            pl.BlockSpec((1, 1, P_PAD), lambda i: (i, 0, 0)),
            pl.BlockSpec((1, 1, 128), lambda i: (i, 0, 0)),
            pl.BlockSpec(memory_space=pltpu.SMEM),
        ],
        out_shape=[
            jax.ShapeDtypeStruct((B, 1, P_PAD), jnp.float32),
            jax.ShapeDtypeStruct((B, 1, 128), jnp.float32),
            jax.ShapeDtypeStruct((1, 4), jnp.float32),
        ],
        scratch_shapes=[pltpu.SMEM((4,), jnp.float32)],
    )(locs_t, scores_t, att, boxes, labels_c, ignored_regions, priors_t)

    topk_rows = _sc_topk(cn.reshape(B, P_PAD), npos[:, 0, 0:16])

    total_pos = tot[0, 0]
    loss = ((tot[0, 2] + jnp.sum(topk_rows[:, 0]) + tot[0, 1]) / total_pos
            + tot[0, 3])
    return loss.reshape(())


# speedup vs baseline: 1.1006x; 1.1006x over previous
import functools

import jax
import jax.numpy as jnp
from jax.experimental import pallas as pl
from jax.experimental.pallas import tpu as pltpu
from jax.experimental.pallas import tpu_sc as plsc

N_PRIORS_C = 21420
P_PAD = 21504
BATCH_C = 8
N_OBJ_C = 16
N_IGN_C = 4
N_CLASSES_C = 4
THRESHOLD_C = 0.4
NEG_POS_RATIO_C = 2
THETA_C = 0.1
ATT_HW = 56 * 96
NCHUNK = P_PAD // 16


def _pairwise_iou(bx1, by1, bx2, by2, px1, py1, px2, py2):
    lt_x = jnp.maximum(bx1, px1)
    lt_y = jnp.maximum(by1, py1)
    rb_x = jnp.minimum(bx2, px2)
    rb_y = jnp.minimum(by2, py2)
    inter = jnp.clip(rb_x - lt_x, 0.0, None) * jnp.clip(rb_y - lt_y, 0.0, None)
    area_b = (bx2 - bx1) * (by2 - by1)
    area_p = (px2 - px1) * (py2 - py1)
    union = area_b + area_p - inter
    return inter / union


def _image_kernel(locs_ref, scores_ref, att_ref, boxes_ref, labels_ref,
                  ign_ref, priors_ref, cn_ref, npos_ref, tot_ref, acc_ref):
    i = pl.program_id(0)

    @pl.when(i == 0)
    def _init():
        acc_ref[0] = 0.0
        acc_ref[1] = 0.0
        acc_ref[2] = 0.0
        acc_ref[3] = 0.0

    lane = jax.lax.broadcasted_iota(jnp.int32, (1, P_PAD), 1)
    lane_valid = lane < N_PRIORS_C

    pcx = priors_ref[0:1, :]
    pcy = priors_ref[1:2, :]
    pw = priors_ref[2:3, :]
    ph = priors_ref[3:4, :]
    px1 = pcx - pw * 0.5
    py1 = pcy - ph * 0.5
    px2 = pcx + pw * 0.5
    py2 = pcy + ph * 0.5

    b = boxes_ref[0]
    bx1 = b[:, 0:1]
    by1 = b[:, 1:2]
    bx2 = b[:, 2:3]
    by2 = b[:, 3:4]

    ov = _pairwise_iou(bx1, by1, bx2, by2, px1, py1, px2, py2)

    iota_obj = jax.lax.broadcasted_iota(jnp.int32, (N_OBJ_C, P_PAD), 0)
    iota_pri = jax.lax.broadcasted_iota(jnp.int32, (N_OBJ_C, P_PAD), 1)

    ofp = jnp.max(ov, axis=0, keepdims=True)
    obj_fp = jnp.min(jnp.where(ov == ofp, iota_obj, N_OBJ_C), axis=0,
                     keepdims=True)

    ofo = jnp.max(ov, axis=1, keepdims=True)
    pfo = jnp.min(jnp.where(ov == ofo, iota_pri, P_PAD), axis=1,
                  keepdims=True)
    valid = ofo > 0.0

    c = valid.astype(jnp.int32)
    for s in (1, 2, 4, 8):
        shifted = jnp.concatenate(
            [jnp.zeros((s, 1), jnp.int32), c[: N_OBJ_C - s, :]], axis=0)
        c = c + shifted
    rank = c - 1

    obj_j = jax.lax.broadcasted_iota(jnp.int32, (N_OBJ_C, 1), 0)
    match = pfo == lane
    j_sel = jnp.max(jnp.where(match, obj_j, -1), axis=0, keepdims=True)
    onehot2 = (j_sel == iota_obj).astype(jnp.float32)
    w2 = jnp.concatenate([valid.astype(jnp.float32),
                          rank.astype(jnp.float32)], axis=1)
    g2 = jax.lax.dot_general(w2, onehot2, (((0,), (0,)), ((), ())),
                             preferred_element_type=jnp.float32)
    valid_sel = g2[0:1, :] >= 0.5
    ofp = jnp.where(valid_sel, 1.0, ofp)
    obj_f = jnp.where(valid_sel, g2[1:2, :], obj_fp.astype(jnp.float32))

    iota_obj_f = iota_obj.astype(jnp.float32)
    onehot = (obj_f == iota_obj_f).astype(jnp.float32)
    labels_col = labels_ref[0].astype(jnp.float32)
    w5 = jnp.concatenate([labels_col, bx1, by1, bx2, by2], axis=1)
    g5 = jax.lax.dot_general(w5, onehot, (((0,), (0,)), ((), ())),
                             preferred_element_type=jnp.float32)
    lab = jnp.where(ofp < THRESHOLD_C, 0.0, g5[0:1, :])
    tx1 = g5[1:2, :]
    ty1 = g5[2:3, :]
    tx2 = g5[3:4, :]
    ty2 = g5[4:5, :]

    pos = lab > 0.0
    posf = pos.astype(jnp.float32)
    n_pos = jnp.sum(posf)

    g = ign_ref[0]
    gx1 = g[:, 0:1]
    gy1 = g[:, 1:2]
    gx2 = g[:, 2:3]
    gy2 = g[:, 3:4]
    i_x = jnp.clip(jnp.minimum(gx2, px2) - jnp.maximum(gx1, px1), 0.0, None)
    i_y = jnp.clip(jnp.minimum(gy2, py2) - jnp.maximum(gy1, py1), 0.0, None)
    inter_g = i_x * i_y
    area_sum = (gx2 - gx1) * (gy2 - gy1) + (px2 - px1) * (py2 - py1)
    ign = jnp.max(jnp.where(11.0 * inter_g >= area_sum, 1, 0), axis=0,
                  keepdims=True) > 0

    gl = locs_ref[0]
    d_cx = gl[0:1, :] * pw / 10.0 + pcx
    d_cy = gl[1:2, :] * ph / 10.0 + pcy
    d_w = jnp.exp(gl[2:3, :] / 5.0) * pw
    d_h = jnp.exp(gl[3:4, :] / 5.0) * ph
    dx1 = d_cx - d_w * 0.5
    dy1 = d_cy - d_h * 0.5
    dx2 = d_cx + d_w * 0.5
    dy2 = d_cy + d_h * 0.5

    ix1 = jnp.maximum(dx1, tx1)
    iy1 = jnp.maximum(dy1, ty1)
    ix2 = jnp.minimum(dx2, tx2)
    iy2 = jnp.minimum(dy2, ty2)
    inter = jnp.clip(ix2 - ix1, 0.0, None) * jnp.clip(iy2 - iy1, 0.0, None)
    ap = (dx2 - dx1) * (dy2 - dy1)
    at = (tx2 - tx1) * (ty2 - ty1)
    union = ap + at - inter
    iou = inter / (union + 1e-9)
    cxp = (dx1 + dx2) * 0.5
    cyp = (dy1 + dy2) * 0.5
    cxt = (tx1 + tx2) * 0.5
    cyt = (ty1 + ty2) * 0.5
    d2 = (cxp - cxt) ** 2 + (cyp - cyt) ** 2
    ex1 = jnp.minimum(dx1, tx1)
    ey1 = jnp.minimum(dy1, ty1)
    ex2 = jnp.maximum(dx2, tx2)
    ey2 = jnp.maximum(dy2, ty2)
    c2 = (ex2 - ex1) ** 2 + (ey2 - ey1) ** 2 + 1e-7
    diou = 1.0 - iou + d2 / c2
    loc_sum = jnp.sum(diou * posf)

    s = scores_ref[0]
    s0 = s[0:1, :]
    s1 = s[1:2, :]
    s2 = s[2:3, :]
    s3 = s[3:4, :]
    m = jnp.maximum(jnp.maximum(s0, s1), jnp.maximum(s2, s3))
    lse = m + jnp.log(jnp.exp(s0 - m) + jnp.exp(s1 - m)
                      + jnp.exp(s2 - m) + jnp.exp(s3 - m))
    picked = jnp.where(lab == 0.0, s0, 0.0) + jnp.where(lab == 1.0, s1, 0.0) \
        + jnp.where(lab == 2.0, s2, 0.0) + jnp.where(lab == 3.0, s3, 0.0)
    ce = lse - picked
    conf_pos_sum = jnp.sum(ce * posf)

    neg_mask = jnp.logical_not(pos | ign) & lane_valid
    cn_ref[0] = jnp.where(neg_mask, ce, 0.0)
    npos_ref[0] = jnp.full((1, 128), n_pos, jnp.float32)

    a = att_ref[0]
    seg = -jnp.sum(jnp.clip(jnp.log(1.0 - a), -100.0, None))

    acc_ref[0] = acc_ref[0] + n_pos
    acc_ref[1] = acc_ref[1] + loc_sum
    acc_ref[2] = acc_ref[2] + conf_pos_sum
    acc_ref[3] = acc_ref[3] + seg

    @pl.when(i == BATCH_C - 1)
    def _fin():
        tot_ref[0, 0] = acc_ref[0]
        tot_ref[0, 1] = acc_ref[1]
        tot_ref[0, 2] = acc_ref[2]
        tot_ref[0, 3] = acc_ref[3]


def _sc_topk_kernel(cn_hbm, npos_hbm, out_hbm, vals, npv, hcnt, hsum, orow,
                    sem):
    wid = jax.lax.axis_index("s") * 2 + jax.lax.axis_index("c")

    @pl.when(wid < BATCH_C)
    def _active():
        img = wid
        pltpu.sync_copy(cn_hbm.at[img], vals)
        pltpu.sync_copy(npos_hbm.at[img], npv)

        lanes = jax.lax.broadcasted_iota(jnp.int32, (16,), 0)
        ones_i = jnp.ones((16,), jnp.int32)
        zeros_i = jnp.zeros((16,), jnp.int32)
        zeros_f = jnp.zeros((16,), jnp.float32)
        lane_base = lanes * 256

        n_pos = jnp.sum(jnp.where(lanes == 0, npv[...], 0.0))
        k = jnp.minimum(
            NEG_POS_RATIO_C * n_pos.astype(jnp.int32), N_PRIORS_C)

        def _zero_hists():
            @plsc.parallel_loop(0, 2 * 256, unroll=8)
            def _z(bb):
                hcnt[pl.ds(bb * 16, 16)] = zeros_i
                hsum[pl.ds(bb * 16, 16)] = zeros_f

        def _suffix_stats(kk):
            ccnt = []
            csum = []
            for j in range(16):
                ac = zeros_i
                asm = zeros_f
                for l in range(32):
                    ac = ac + hcnt[pl.ds(l * 256 + j * 16, 16)]
                    asm = asm + hsum[pl.ds(l * 256 + j * 16, 16)]
                ccnt.append(ac)
                csum.append(asm)
            carry = jnp.int32(0)
            nb = jnp.int32(0)
            sgrp = [None] * 16
            for j in range(15, -1, -1):
                r = jax.lax.rev(ccnt[j], (0,))
                sfx = jax.lax.rev(jnp.cumsum(r), (0,)) + carry
                sgrp[j] = sfx
                carry = carry + jnp.sum(ccnt[j])
            for j in range(16):
                nb = nb + jnp.sum((sgrp[j] >= kk).astype(jnp.int32))
            bstar = nb - 1
            cnt_above = jnp.int32(0)
            sum_above = jnp.float32(0.0)
            bcnt = jnp.int32(0)
            bsum = jnp.float32(0.0)
            for j in range(16):
                binv = lanes + j * 16
                above = binv > bstar
                at_b = binv == bstar
                cnt_above = cnt_above + jnp.sum(
                    jnp.where(above, ccnt[j], 0))
                sum_above = sum_above + jnp.sum(
                    jnp.where(above, csum[j], 0.0))
                bcnt = bcnt + jnp.sum(jnp.where(at_b, ccnt[j], 0))
                bsum = bsum + jnp.sum(jnp.where(at_b, csum[j], 0.0))
            return bstar, cnt_above, sum_above, bcnt, bsum

        _zero_hists()

        @plsc.parallel_loop(0, NCHUNK, unroll=2)
        def pass1(cc):
            v = vals[pl.ds(cc * 16, 16)]
            bits = plsc.bitcast(v, jnp.int32)
            copy = (cc & 1) * 4096
            idx = jax.lax.shift_right_logical(bits, 23) + lane_base + copy
            plsc.addupdate_scatter(hcnt, [idx], ones_i)
            plsc.addupdate_scatter(hsum, [idx], v)

        bstar, cnt_ab1, sum_ab1, _, _ = _suffix_stats(k)

        _zero_hists()
        k2 = k - cnt_ab1

        @plsc.parallel_loop(0, NCHUNK, unroll=2)
        def pass2(cc):
            v = vals[pl.ds(cc * 16, 16)]
            bits = plsc.bitcast(v, jnp.int32)
            coarse = jax.lax.shift_right_logical(bits, 23)
            fine = jax.lax.shift_right_logical(bits, 15) & 0xFF
            msk = coarse == bstar
            copy = (cc & 1) * 4096
            idx = fine + lane_base + copy
            plsc.addupdate_scatter(hcnt, [idx], ones_i, mask=msk)
            plsc.addupdate_scatter(hsum, [idx], v, mask=msk)

        _, cnt_ab2, sum_ab2, bcnt, bsum = _suffix_stats(k2)

        ties_v = jnp.full((16,), (k2 - cnt_ab2).astype(jnp.float32))
        bsum_v = jnp.full((16,), bsum)
        bcnt_v = jnp.maximum(jnp.full((16,), bcnt.astype(jnp.float32)), 1.0)
        t_v = bsum_v / bcnt_v
        sum_v = jnp.full((16,), sum_ab1 + sum_ab2)
        k_v = jnp.full((16,), k)
        topk_v = jnp.where(k_v > 0, sum_v + ties_v * t_v,
                           jnp.zeros((16,), jnp.float32))
        orow[...] = topk_v
        pltpu.sync_copy(orow, out_hbm.at[img])


_sc_topk = functools.partial(
    pl.kernel,
    mesh=plsc.VectorSubcoreMesh(core_axis_name="c", subcore_axis_name="s"),
    out_type=jax.ShapeDtypeStruct((BATCH_C, 16), jnp.float32),
    scratch_types=[
        pltpu.VMEM((P_PAD,), jnp.float32),
        pltpu.VMEM((16,), jnp.float32),
        pltpu.VMEM((8192,), jnp.int32),
        pltpu.VMEM((8192,), jnp.float32),
        pltpu.VMEM((16,), jnp.float32),
        pltpu.SemaphoreType.DMA,
    ],
    compiler_params=pltpu.CompilerParams(needs_layout_passes=False),
)(_sc_topk_kernel)


@jax.jit
def kernel(odm_locs, odm_scores, attention_map, boxes, labels,
           ignored_regions, priors_cxcy):
    B, P, _ = odm_scores.shape
    pad = P_PAD - P

    locs_t = jnp.pad(jnp.transpose(odm_locs, (0, 2, 1)),
                     ((0, 0), (0, 0), (0, pad)))
    scores_t = jnp.pad(jnp.transpose(odm_scores, (0, 2, 1)),
                       ((0, 0), (0, 0), (0, pad)))
    sentinel = jnp.tile(jnp.array([[-9.0], [-9.0], [1.0], [1.0]],
                                  jnp.float32), (1, pad))
    priors_t = jnp.concatenate(
        [jnp.transpose(priors_cxcy, (1, 0)), sentinel], axis=1)
    att = attention_map.reshape(B, 1, ATT_HW)
    labels_c = labels.astype(jnp.int32).reshape(B, N_OBJ_C, 1)

    cn, npos, tot = pl.pallas_call(
        _image_kernel,
        grid=(B,),
        in_specs=[
            pl.BlockSpec((1, 4, P_PAD), lambda i: (i, 0, 0)),
            pl.BlockSpec((1, 4, P_PAD), lambda i: (i, 0, 0)),
            pl.BlockSpec((1, 1, ATT_HW), lambda i: (i, 0, 0)),
            pl.BlockSpec((1, N_OBJ_C, 4), lambda i: (i, 0, 0)),
            pl.BlockSpec((1, N_OBJ_C, 1), lambda i: (i, 0, 0)),
            pl.BlockSpec((1, N_IGN_C, 4), lambda i: (i, 0, 0)),
            pl.BlockSpec((4, P_PAD), lambda i: (0, 0)),
        ],
        out_specs=[
            pl.BlockSpec((1, 1, P_PAD), lambda i: (i, 0, 0)),
            pl.BlockSpec((1, 1, 128), lambda i: (i, 0, 0)),
            pl.BlockSpec(memory_space=pltpu.SMEM),
        ],
        out_shape=[
            jax.ShapeDtypeStruct((B, 1, P_PAD), jnp.float32),
            jax.ShapeDtypeStruct((B, 1, 128), jnp.float32),
            jax.ShapeDtypeStruct((1, 4), jnp.float32),
        ],
        scratch_shapes=[pltpu.SMEM((4,), jnp.float32)],
    )(locs_t, scores_t, att, boxes, labels_c, ignored_regions, priors_t)

    topk_rows = _sc_topk(cn.reshape(B, P_PAD), npos[:, 0, 0:16])

    total_pos = tot[0, 0]
    loss = ((tot[0, 2] + jnp.sum(topk_rows[:, 0]) + tot[0, 1]) / total_pos
            + tot[0, 3])
    return loss.reshape(())


# speedup vs baseline: 1.2143x; 1.1033x over previous
import functools

import jax
import jax.numpy as jnp
from jax.experimental import pallas as pl
from jax.experimental.pallas import tpu as pltpu
from jax.experimental.pallas import tpu_sc as plsc

N_PRIORS_C = 21420
P_PAD = 21504
BATCH_C = 8
N_OBJ_C = 16
N_IGN_C = 4
N_CLASSES_C = 4
THRESHOLD_C = 0.4
NEG_POS_RATIO_C = 2
THETA_C = 0.1
ATT_HW = 56 * 96
NCHUNK = P_PAD // 16


def _pairwise_iou(bx1, by1, bx2, by2, px1, py1, px2, py2):
    lt_x = jnp.maximum(bx1, px1)
    lt_y = jnp.maximum(by1, py1)
    rb_x = jnp.minimum(bx2, px2)
    rb_y = jnp.minimum(by2, py2)
    inter = jnp.clip(rb_x - lt_x, 0.0, None) * jnp.clip(rb_y - lt_y, 0.0, None)
    area_b = (bx2 - bx1) * (by2 - by1)
    area_p = (px2 - px1) * (py2 - py1)
    union = area_b + area_p - inter
    return inter / union


def _image_kernel(scores_ref, boxes_ref, labels_ref,
                  ign_ref, priors_ref, cn_ref, npos_ref, obj_ref, posf_ref,
                  tot_ref, acc_ref):
    i = pl.program_id(0)

    @pl.when(i == 0)
    def _init():
        acc_ref[0] = 0.0
        acc_ref[1] = 0.0

    lane = jax.lax.broadcasted_iota(jnp.int32, (1, P_PAD), 1)
    lane_valid = lane < N_PRIORS_C

    pcx = priors_ref[0:1, :]
    pcy = priors_ref[1:2, :]
    pw = priors_ref[2:3, :]
    ph = priors_ref[3:4, :]
    px1 = pcx - pw * 0.5
    py1 = pcy - ph * 0.5
    px2 = pcx + pw * 0.5
    py2 = pcy + ph * 0.5

    b = boxes_ref[0]
    bx1 = b[:, 0:1]
    by1 = b[:, 1:2]
    bx2 = b[:, 2:3]
    by2 = b[:, 3:4]

    ov = _pairwise_iou(bx1, by1, bx2, by2, px1, py1, px2, py2)

    iota_obj = jax.lax.broadcasted_iota(jnp.int32, (N_OBJ_C, P_PAD), 0)
    iota_pri = jax.lax.broadcasted_iota(jnp.int32, (N_OBJ_C, P_PAD), 1)

    ofp = jnp.max(ov, axis=0, keepdims=True)
    obj_fp = jnp.min(jnp.where(ov == ofp, iota_obj, N_OBJ_C), axis=0,
                     keepdims=True)

    ofo = jnp.max(ov, axis=1, keepdims=True)
    pfo = jnp.min(jnp.where(ov == ofo, iota_pri, P_PAD), axis=1,
                  keepdims=True)
    valid = ofo > 0.0

    c = valid.astype(jnp.int32)
    for s in (1, 2, 4, 8):
        shifted = jnp.concatenate(
            [jnp.zeros((s, 1), jnp.int32), c[: N_OBJ_C - s, :]], axis=0)
        c = c + shifted
    rank = c - 1

    obj_j = jax.lax.broadcasted_iota(jnp.int32, (N_OBJ_C, 1), 0)
    match = pfo == lane
    j_sel = jnp.max(jnp.where(match, obj_j, -1), axis=0, keepdims=True)
    onehot2 = (j_sel == iota_obj).astype(jnp.float32)
    w2 = jnp.concatenate([valid.astype(jnp.float32),
                          rank.astype(jnp.float32)], axis=1)
    g2 = jax.lax.dot_general(w2, onehot2, (((0,), (0,)), ((), ())),
                             preferred_element_type=jnp.float32)
    valid_sel = g2[0:1, :] >= 0.5
    ofp = jnp.where(valid_sel, 1.0, ofp)
    obj_f = jnp.where(valid_sel, g2[1:2, :], obj_fp.astype(jnp.float32))

    iota_obj_f = iota_obj.astype(jnp.float32)
    onehot = (obj_f == iota_obj_f).astype(jnp.float32)
    labels_col = labels_ref[0].astype(jnp.float32)
    g5 = jax.lax.dot_general(labels_col, onehot, (((0,), (0,)), ((), ())),
                             preferred_element_type=jnp.float32)
    lab = jnp.where(ofp < THRESHOLD_C, 0.0, g5[0:1, :])

    pos = lab > 0.0
    posf = pos.astype(jnp.float32)
    n_pos = jnp.sum(posf)

    g = ign_ref[0]
    gx1 = g[:, 0:1]
    gy1 = g[:, 1:2]
    gx2 = g[:, 2:3]
    gy2 = g[:, 3:4]
    i_x = jnp.clip(jnp.minimum(gx2, px2) - jnp.maximum(gx1, px1), 0.0, None)
    i_y = jnp.clip(jnp.minimum(gy2, py2) - jnp.maximum(gy1, py1), 0.0, None)
    inter_g = i_x * i_y
    area_sum = (gx2 - gx1) * (gy2 - gy1) + (px2 - px1) * (py2 - py1)
    ign = jnp.max(jnp.where(11.0 * inter_g >= area_sum, 1, 0), axis=0,
                  keepdims=True) > 0

    s = scores_ref[0]
    s0 = s[0:1, :]
    s1 = s[1:2, :]
    s2 = s[2:3, :]
    s3 = s[3:4, :]
    m = jnp.maximum(jnp.maximum(s0, s1), jnp.maximum(s2, s3))
    lse = m + jnp.log(jnp.exp(s0 - m) + jnp.exp(s1 - m)
                      + jnp.exp(s2 - m) + jnp.exp(s3 - m))
    picked = jnp.where(lab == 0.0, s0, 0.0) + jnp.where(lab == 1.0, s1, 0.0) \
        + jnp.where(lab == 2.0, s2, 0.0) + jnp.where(lab == 3.0, s3, 0.0)
    ce = lse - picked
    conf_pos_sum = jnp.sum(ce * posf)

    neg_mask = jnp.logical_not(pos | ign) & lane_valid
    cn_ref[0] = jnp.where(neg_mask, ce, 0.0)
    npos_ref[0] = jnp.full((1, 128), n_pos, jnp.float32)
    obj_ref[0] = obj_f
    posf_ref[0] = posf

    acc_ref[0] = acc_ref[0] + n_pos
    acc_ref[1] = acc_ref[1] + conf_pos_sum

    @pl.when(i == BATCH_C - 1)
    def _fin():
        tot_ref[0, 0] = acc_ref[0]
        tot_ref[0, 1] = acc_ref[1]


def _decode_kernel(locs_ref, att_ref, boxes_ref, priors_ref, obj_ref,
                   posf_ref, tot_ref, acc_ref):
    i = pl.program_id(0)

    @pl.when(i == 0)
    def _init():
        acc_ref[0] = 0.0
        acc_ref[1] = 0.0

    pcx = priors_ref[0:1, :]
    pcy = priors_ref[1:2, :]
    pw = priors_ref[2:3, :]
    ph = priors_ref[3:4, :]

    b = boxes_ref[0]
    obj_f = obj_ref[0]
    posf = posf_ref[0]
    iota_obj_f = jax.lax.broadcasted_iota(
        jnp.int32, (N_OBJ_C, P_PAD), 0).astype(jnp.float32)
    onehot = (obj_f == iota_obj_f).astype(jnp.float32)
    g4 = jax.lax.dot_general(b, onehot, (((0,), (0,)), ((), ())),
                             preferred_element_type=jnp.float32)
    tx1 = g4[0:1, :]
    ty1 = g4[1:2, :]
    tx2 = g4[2:3, :]
    ty2 = g4[3:4, :]

    gl = locs_ref[0]
    d_cx = gl[0:1, :] * pw / 10.0 + pcx
    d_cy = gl[1:2, :] * ph / 10.0 + pcy
    d_w = jnp.exp(gl[2:3, :] / 5.0) * pw
    d_h = jnp.exp(gl[3:4, :] / 5.0) * ph
    dx1 = d_cx - d_w * 0.5
    dy1 = d_cy - d_h * 0.5
    dx2 = d_cx + d_w * 0.5
    dy2 = d_cy + d_h * 0.5

    ix1 = jnp.maximum(dx1, tx1)
    iy1 = jnp.maximum(dy1, ty1)
    ix2 = jnp.minimum(dx2, tx2)
    iy2 = jnp.minimum(dy2, ty2)
    inter = jnp.clip(ix2 - ix1, 0.0, None) * jnp.clip(iy2 - iy1, 0.0, None)
    ap = (dx2 - dx1) * (dy2 - dy1)
    at = (tx2 - tx1) * (ty2 - ty1)
    union = ap + at - inter
    iou = inter / (union + 1e-9)
    cxp = (dx1 + dx2) * 0.5
    cyp = (dy1 + dy2) * 0.5
    cxt = (tx1 + tx2) * 0.5
    cyt = (ty1 + ty2) * 0.5
    d2 = (cxp - cxt) ** 2 + (cyp - cyt) ** 2
    ex1 = jnp.minimum(dx1, tx1)
    ey1 = jnp.minimum(dy1, ty1)
    ex2 = jnp.maximum(dx2, tx2)
    ey2 = jnp.maximum(dy2, ty2)
    c2 = (ex2 - ex1) ** 2 + (ey2 - ey1) ** 2 + 1e-7
    diou = 1.0 - iou + d2 / c2
    loc_sum = jnp.sum(diou * posf)

    a = att_ref[0]
    seg = -jnp.sum(jnp.clip(jnp.log(1.0 - a), -100.0, None))

    acc_ref[0] = acc_ref[0] + loc_sum
    acc_ref[1] = acc_ref[1] + seg

    @pl.when(i == BATCH_C - 1)
    def _fin():
        tot_ref[0, 0] = acc_ref[0]
        tot_ref[0, 1] = acc_ref[1]


def _sc_topk_kernel(cn_hbm, npos_hbm, out_hbm, vals, npv, hcnt, hsum, orow,
                    sem):
    wid = jax.lax.axis_index("s") * 2 + jax.lax.axis_index("c")

    @pl.when(wid < BATCH_C)
    def _active():
        img = wid
        pltpu.sync_copy(cn_hbm.at[img], vals)
        pltpu.sync_copy(npos_hbm.at[img], npv)

        lanes = jax.lax.broadcasted_iota(jnp.int32, (16,), 0)
        ones_i = jnp.ones((16,), jnp.int32)
        zeros_i = jnp.zeros((16,), jnp.int32)
        zeros_f = jnp.zeros((16,), jnp.float32)
        lane_base = lanes * 256

        n_pos = jnp.sum(jnp.where(lanes == 0, npv[...], 0.0))
        k = jnp.minimum(
            NEG_POS_RATIO_C * n_pos.astype(jnp.int32), N_PRIORS_C)

        def _zero_hists():
            @plsc.parallel_loop(0, 2 * 256, unroll=8)
            def _z(bb):
                hcnt[pl.ds(bb * 16, 16)] = zeros_i
                hsum[pl.ds(bb * 16, 16)] = zeros_f

        def _suffix_stats(kk):
            ccnt = []
            csum = []
            for j in range(16):
                ac = zeros_i
                asm = zeros_f
                for l in range(32):
                    ac = ac + hcnt[pl.ds(l * 256 + j * 16, 16)]
                    asm = asm + hsum[pl.ds(l * 256 + j * 16, 16)]
                ccnt.append(ac)
                csum.append(asm)
            carry = jnp.int32(0)
            nb = jnp.int32(0)
            sgrp = [None] * 16
            for j in range(15, -1, -1):
                r = jax.lax.rev(ccnt[j], (0,))
                sfx = jax.lax.rev(jnp.cumsum(r), (0,)) + carry
                sgrp[j] = sfx
                carry = carry + jnp.sum(ccnt[j])
            for j in range(16):
                nb = nb + jnp.sum((sgrp[j] >= kk).astype(jnp.int32))
            bstar = nb - 1
            cnt_above = jnp.int32(0)
            sum_above = jnp.float32(0.0)
            bcnt = jnp.int32(0)
            bsum = jnp.float32(0.0)
            for j in range(16):
                binv = lanes + j * 16
                above = binv > bstar
                at_b = binv == bstar
                cnt_above = cnt_above + jnp.sum(
                    jnp.where(above, ccnt[j], 0))
                sum_above = sum_above + jnp.sum(
                    jnp.where(above, csum[j], 0.0))
                bcnt = bcnt + jnp.sum(jnp.where(at_b, ccnt[j], 0))
                bsum = bsum + jnp.sum(jnp.where(at_b, csum[j], 0.0))
            return bstar, cnt_above, sum_above, bcnt, bsum

        _zero_hists()

        @plsc.parallel_loop(0, NCHUNK, unroll=2)
        def pass1(cc):
            v = vals[pl.ds(cc * 16, 16)]
            bits = plsc.bitcast(v, jnp.int32)
            copy = (cc & 1) * 4096
            idx = jax.lax.shift_right_logical(bits, 23) + lane_base + copy
            plsc.addupdate_scatter(hcnt, [idx], ones_i)
            plsc.addupdate_scatter(hsum, [idx], v)

        bstar, cnt_ab1, sum_ab1, _, _ = _suffix_stats(k)

        _zero_hists()
        k2 = k - cnt_ab1

        @plsc.parallel_loop(0, NCHUNK, unroll=2)
        def pass2(cc):
            v = vals[pl.ds(cc * 16, 16)]
            bits = plsc.bitcast(v, jnp.int32)
            coarse = jax.lax.shift_right_logical(bits, 23)
            fine = jax.lax.shift_right_logical(bits, 15) & 0xFF
            msk = coarse == bstar
            copy = (cc & 1) * 4096
            idx = fine + lane_base + copy
            plsc.addupdate_scatter(hcnt, [idx], ones_i, mask=msk)
            plsc.addupdate_scatter(hsum, [idx], v, mask=msk)

        _, cnt_ab2, sum_ab2, bcnt, bsum = _suffix_stats(k2)

        ties_v = jnp.full((16,), (k2 - cnt_ab2).astype(jnp.float32))
        bsum_v = jnp.full((16,), bsum)
        bcnt_v = jnp.maximum(jnp.full((16,), bcnt.astype(jnp.float32)), 1.0)
        t_v = bsum_v / bcnt_v
        sum_v = jnp.full((16,), sum_ab1 + sum_ab2)
        k_v = jnp.full((16,), k)
        topk_v = jnp.where(k_v > 0, sum_v + ties_v * t_v,
                           jnp.zeros((16,), jnp.float32))
        orow[...] = topk_v
        pltpu.sync_copy(orow, out_hbm.at[img])


_sc_topk = functools.partial(
    pl.kernel,
    mesh=plsc.VectorSubcoreMesh(core_axis_name="c", subcore_axis_name="s"),
    out_type=jax.ShapeDtypeStruct((BATCH_C, 16), jnp.float32),
    scratch_types=[
        pltpu.VMEM((P_PAD,), jnp.float32),
        pltpu.VMEM((16,), jnp.float32),
        pltpu.VMEM((8192,), jnp.int32),
        pltpu.VMEM((8192,), jnp.float32),
        pltpu.VMEM((16,), jnp.float32),
        pltpu.SemaphoreType.DMA,
    ],
    compiler_params=pltpu.CompilerParams(needs_layout_passes=False),
)(_sc_topk_kernel)


@jax.jit
def kernel(odm_locs, odm_scores, attention_map, boxes, labels,
           ignored_regions, priors_cxcy):
    B, P, _ = odm_scores.shape
    pad = P_PAD - P

    locs_t = jnp.pad(jnp.transpose(odm_locs, (0, 2, 1)),
                     ((0, 0), (0, 0), (0, pad)))
    scores_t = jnp.pad(jnp.transpose(odm_scores, (0, 2, 1)),
                       ((0, 0), (0, 0), (0, pad)))
    sentinel = jnp.tile(jnp.array([[-9.0], [-9.0], [1.0], [1.0]],
                                  jnp.float32), (1, pad))
    priors_t = jnp.concatenate(
        [jnp.transpose(priors_cxcy, (1, 0)), sentinel], axis=1)
    att = attention_map.reshape(B, 1, ATT_HW)
    labels_c = labels.astype(jnp.int32).reshape(B, N_OBJ_C, 1)

    cn, npos, obj, posf, tot = pl.pallas_call(
        _image_kernel,
        grid=(B,),
        in_specs=[
            pl.BlockSpec((1, 4, P_PAD), lambda i: (i, 0, 0)),
            pl.BlockSpec((1, N_OBJ_C, 4), lambda i: (i, 0, 0)),
            pl.BlockSpec((1, N_OBJ_C, 1), lambda i: (i, 0, 0)),
            pl.BlockSpec((1, N_IGN_C, 4), lambda i: (i, 0, 0)),
            pl.BlockSpec((4, P_PAD), lambda i: (0, 0)),
        ],
        out_specs=[
            pl.BlockSpec((1, 1, P_PAD), lambda i: (i, 0, 0)),
            pl.BlockSpec((1, 1, 128), lambda i: (i, 0, 0)),
            pl.BlockSpec((1, 1, P_PAD), lambda i: (i, 0, 0)),
            pl.BlockSpec((1, 1, P_PAD), lambda i: (i, 0, 0)),
            pl.BlockSpec(memory_space=pltpu.SMEM),
        ],
        out_shape=[
            jax.ShapeDtypeStruct((B, 1, P_PAD), jnp.float32),
            jax.ShapeDtypeStruct((B, 1, 128), jnp.float32),
            jax.ShapeDtypeStruct((B, 1, P_PAD), jnp.float32),
            jax.ShapeDtypeStruct((B, 1, P_PAD), jnp.float32),
            jax.ShapeDtypeStruct((1, 2), jnp.float32),
        ],
        scratch_shapes=[pltpu.SMEM((2,), jnp.float32)],
    )(scores_t, boxes, labels_c, ignored_regions, priors_t)

    topk_rows = _sc_topk(cn.reshape(B, P_PAD), npos[:, 0, 0:16])

    tot2 = pl.pallas_call(
        _decode_kernel,
        grid=(B,),
        in_specs=[
            pl.BlockSpec((1, 4, P_PAD), lambda i: (i, 0, 0)),
            pl.BlockSpec((1, 1, ATT_HW), lambda i: (i, 0, 0)),
            pl.BlockSpec((1, N_OBJ_C, 4), lambda i: (i, 0, 0)),
            pl.BlockSpec((4, P_PAD), lambda i: (0, 0)),
            pl.BlockSpec((1, 1, P_PAD), lambda i: (i, 0, 0)),
            pl.BlockSpec((1, 1, P_PAD), lambda i: (i, 0, 0)),
        ],
        out_specs=pl.BlockSpec(memory_space=pltpu.SMEM),
        out_shape=jax.ShapeDtypeStruct((1, 2), jnp.float32),
        scratch_shapes=[pltpu.SMEM((2,), jnp.float32)],
    )(locs_t, att, boxes, priors_t, obj, posf)

    total_pos = tot[0, 0]
    loss = ((tot[0, 1] + jnp.sum(topk_rows[:, 0]) + tot2[0, 0]) / total_pos
            + tot2[0, 1])
    return loss.reshape(())


# speedup vs baseline: 1.2482x; 1.0279x over previous
import functools

import jax
import jax.numpy as jnp
from jax.experimental import pallas as pl
from jax.experimental.pallas import tpu as pltpu
from jax.experimental.pallas import tpu_sc as plsc

N_PRIORS_C = 21420
P_PAD = 21504
BATCH_C = 8
N_OBJ_C = 16
N_IGN_C = 4
N_CLASSES_C = 4
THRESHOLD_C = 0.4
NEG_POS_RATIO_C = 2
THETA_C = 0.1
ATT_HW = 56 * 96
NCHUNK = P_PAD // 16


def _pairwise_iou(bx1, by1, bx2, by2, px1, py1, px2, py2):
    lt_x = jnp.maximum(bx1, px1)
    lt_y = jnp.maximum(by1, py1)
    rb_x = jnp.minimum(bx2, px2)
    rb_y = jnp.minimum(by2, py2)
    inter = jnp.clip(rb_x - lt_x, 0.0, None) * jnp.clip(rb_y - lt_y, 0.0, None)
    area_b = (bx2 - bx1) * (by2 - by1)
    area_p = (px2 - px1) * (py2 - py1)
    union = area_b + area_p - inter
    return inter / union


def _image_kernel(scores_ref, boxes_ref, labels_ref,
                  ign_ref, priors_ref, cn_ref, npos_ref, obj_ref, posf_ref,
                  tot_ref, acc_ref):
    i = pl.program_id(0)

    @pl.when(i == 0)
    def _init():
        acc_ref[0] = 0.0
        acc_ref[1] = 0.0

    lane = jax.lax.broadcasted_iota(jnp.int32, (1, P_PAD), 1)
    lane_valid = lane < N_PRIORS_C

    pcx = priors_ref[0:1, :]
    pcy = priors_ref[1:2, :]
    pw = priors_ref[2:3, :]
    ph = priors_ref[3:4, :]
    px1 = pcx - pw * 0.5
    py1 = pcy - ph * 0.5
    px2 = pcx + pw * 0.5
    py2 = pcy + ph * 0.5

    b = boxes_ref[0]
    bx1 = b[:, 0:1]
    by1 = b[:, 1:2]
    bx2 = b[:, 2:3]
    by2 = b[:, 3:4]

    ov = _pairwise_iou(bx1, by1, bx2, by2, px1, py1, px2, py2)

    iota_obj = jax.lax.broadcasted_iota(jnp.int32, (N_OBJ_C, P_PAD), 0)
    iota_pri = jax.lax.broadcasted_iota(jnp.int32, (N_OBJ_C, P_PAD), 1)

    ofp = jnp.max(ov, axis=0, keepdims=True)
    obj_fp = jnp.min(jnp.where(ov == ofp, iota_obj, N_OBJ_C), axis=0,
                     keepdims=True)

    ofo = jnp.max(ov, axis=1, keepdims=True)
    pfo = jnp.min(jnp.where(ov == ofo, iota_pri, P_PAD), axis=1,
                  keepdims=True)
    valid = ofo > 0.0

    c = valid.astype(jnp.int32)
    for s in (1, 2, 4, 8):
        shifted = jnp.concatenate(
            [jnp.zeros((s, 1), jnp.int32), c[: N_OBJ_C - s, :]], axis=0)
        c = c + shifted
    rank = c - 1

    obj_j = jax.lax.broadcasted_iota(jnp.int32, (N_OBJ_C, 1), 0)
    match = pfo == lane
    j_sel = jnp.max(jnp.where(match, obj_j, -1), axis=0, keepdims=True)
    onehot2 = (j_sel == iota_obj).astype(jnp.float32)
    w2 = jnp.concatenate([valid.astype(jnp.float32),
                          rank.astype(jnp.float32)], axis=1)
    g2 = jax.lax.dot_general(w2, onehot2, (((0,), (0,)), ((), ())),
                             preferred_element_type=jnp.float32)
    valid_sel = g2[0:1, :] >= 0.5
    ofp = jnp.where(valid_sel, 1.0, ofp)
    obj_f = jnp.where(valid_sel, g2[1:2, :], obj_fp.astype(jnp.float32))

    iota_obj_f = iota_obj.astype(jnp.float32)
    onehot = (obj_f == iota_obj_f).astype(jnp.float32)
    labels_col = labels_ref[0].astype(jnp.float32)
    g5 = jax.lax.dot_general(labels_col, onehot, (((0,), (0,)), ((), ())),
                             preferred_element_type=jnp.float32)
    lab = jnp.where(ofp < THRESHOLD_C, 0.0, g5[0:1, :])

    pos = lab > 0.0
    posf = pos.astype(jnp.float32)
    n_pos = jnp.sum(posf)

    g = ign_ref[0]
    gx1 = g[:, 0:1]
    gy1 = g[:, 1:2]
    gx2 = g[:, 2:3]
    gy2 = g[:, 3:4]
    i_x = jnp.clip(jnp.minimum(gx2, px2) - jnp.maximum(gx1, px1), 0.0, None)
    i_y = jnp.clip(jnp.minimum(gy2, py2) - jnp.maximum(gy1, py1), 0.0, None)
    inter_g = i_x * i_y
    area_sum = (gx2 - gx1) * (gy2 - gy1) + (px2 - px1) * (py2 - py1)
    ign = jnp.max(jnp.where(11.0 * inter_g >= area_sum, 1, 0), axis=0,
                  keepdims=True) > 0

    s = scores_ref[0]
    s0 = s[0:1, :]
    s1 = s[1:2, :]
    s2 = s[2:3, :]
    s3 = s[3:4, :]
    m = jnp.maximum(jnp.maximum(s0, s1), jnp.maximum(s2, s3))
    lse = m + jnp.log(jnp.exp(s0 - m) + jnp.exp(s1 - m)
                      + jnp.exp(s2 - m) + jnp.exp(s3 - m))
    picked = jnp.where(lab == 0.0, s0, 0.0) + jnp.where(lab == 1.0, s1, 0.0) \
        + jnp.where(lab == 2.0, s2, 0.0) + jnp.where(lab == 3.0, s3, 0.0)
    ce = lse - picked
    conf_pos_sum = jnp.sum(ce * posf)

    neg_mask = jnp.logical_not(pos | ign) & lane_valid
    cn_ref[0] = jnp.where(neg_mask, ce, 0.0)
    npos_ref[0] = jnp.full((1, 128), n_pos, jnp.float32)
    obj_ref[0] = obj_f
    posf_ref[0] = posf

    acc_ref[0] = acc_ref[0] + n_pos
    acc_ref[1] = acc_ref[1] + conf_pos_sum

    @pl.when(i == BATCH_C - 1)
    def _fin():
        tot_ref[0, 0] = acc_ref[0]
        tot_ref[0, 1] = acc_ref[1]


def _decode_kernel(locs_ref, att_ref, boxes_ref, priors_ref, obj_ref,
                   posf_ref, tot_ref, acc_ref):
    i = pl.program_id(0)

    @pl.when(i == 0)
    def _init():
        acc_ref[0] = 0.0
        acc_ref[1] = 0.0

    pcx = priors_ref[0:1, :]
    pcy = priors_ref[1:2, :]
    pw = priors_ref[2:3, :]
    ph = priors_ref[3:4, :]

    b = boxes_ref[0]
    obj_f = obj_ref[0]
    posf = posf_ref[0]
    iota_obj_f = jax.lax.broadcasted_iota(
        jnp.int32, (N_OBJ_C, P_PAD), 0).astype(jnp.float32)
    onehot = (obj_f == iota_obj_f).astype(jnp.float32)
    g4 = jax.lax.dot_general(b, onehot, (((0,), (0,)), ((), ())),
                             preferred_element_type=jnp.float32)
    tx1 = g4[0:1, :]
    ty1 = g4[1:2, :]
    tx2 = g4[2:3, :]
    ty2 = g4[3:4, :]

    gl = locs_ref[0]
    d_cx = gl[0:1, :] * pw / 10.0 + pcx
    d_cy = gl[1:2, :] * ph / 10.0 + pcy
    d_w = jnp.exp(gl[2:3, :] / 5.0) * pw
    d_h = jnp.exp(gl[3:4, :] / 5.0) * ph
    dx1 = d_cx - d_w * 0.5
    dy1 = d_cy - d_h * 0.5
    dx2 = d_cx + d_w * 0.5
    dy2 = d_cy + d_h * 0.5

    ix1 = jnp.maximum(dx1, tx1)
    iy1 = jnp.maximum(dy1, ty1)
    ix2 = jnp.minimum(dx2, tx2)
    iy2 = jnp.minimum(dy2, ty2)
    inter = jnp.clip(ix2 - ix1, 0.0, None) * jnp.clip(iy2 - iy1, 0.0, None)
    ap = (dx2 - dx1) * (dy2 - dy1)
    at = (tx2 - tx1) * (ty2 - ty1)
    union = ap + at - inter
    iou = inter / (union + 1e-9)
    cxp = (dx1 + dx2) * 0.5
    cyp = (dy1 + dy2) * 0.5
    cxt = (tx1 + tx2) * 0.5
    cyt = (ty1 + ty2) * 0.5
    d2 = (cxp - cxt) ** 2 + (cyp - cyt) ** 2
    ex1 = jnp.minimum(dx1, tx1)
    ey1 = jnp.minimum(dy1, ty1)
    ex2 = jnp.maximum(dx2, tx2)
    ey2 = jnp.maximum(dy2, ty2)
    c2 = (ex2 - ex1) ** 2 + (ey2 - ey1) ** 2 + 1e-7
    diou = 1.0 - iou + d2 / c2
    loc_sum = jnp.sum(diou * posf)

    a = att_ref[0]
    seg = -jnp.sum(jnp.clip(jnp.log(1.0 - a), -100.0, None))

    acc_ref[0] = acc_ref[0] + loc_sum
    acc_ref[1] = acc_ref[1] + seg

    @pl.when(i == BATCH_C - 1)
    def _fin():
        tot_ref[0, 0] = acc_ref[0]
        tot_ref[0, 1] = acc_ref[1]


def _sc_topk_kernel(cn_hbm, npos_hbm, out_hbm, vals, npv, hcnt, hsum, orow,
                    sem):
    wid = jax.lax.axis_index("s") * 2 + jax.lax.axis_index("c")

    @pl.when(wid < BATCH_C)
    def _active():
        img = wid
        pltpu.sync_copy(cn_hbm.at[img], vals)
        pltpu.sync_copy(npos_hbm.at[img], npv)

        lanes = jax.lax.broadcasted_iota(jnp.int32, (16,), 0)
        ones_i = jnp.ones((16,), jnp.int32)
        zeros_i = jnp.zeros((16,), jnp.int32)
        zeros_f = jnp.zeros((16,), jnp.float32)
        lane_base = lanes * 256

        n_pos = jnp.sum(jnp.where(lanes == 0, npv[...], 0.0))
        k = jnp.minimum(
            NEG_POS_RATIO_C * n_pos.astype(jnp.int32), N_PRIORS_C)

        def _zero_hists():
            @plsc.parallel_loop(0, 4 * 256, unroll=8)
            def _z(bb):
                hcnt[pl.ds(bb * 16, 16)] = zeros_i
                hsum[pl.ds(bb * 16, 16)] = zeros_f

        def _suffix_stats(kk):
            ccnt = []
            csum = []
            for j in range(16):
                def cbody(l, carry):
                    ac, asm = carry
                    off = l * 256 + j * 16
                    return (ac + hcnt[pl.ds(off, 16)],
                            asm + hsum[pl.ds(off, 16)])
                ac, asm = jax.lax.fori_loop(0, 64, cbody,
                                            (zeros_i, zeros_f))
                ccnt.append(ac)
                csum.append(asm)
            carry = jnp.int32(0)
            nb = jnp.int32(0)
            sgrp = [None] * 16
            for j in range(15, -1, -1):
                r = jax.lax.rev(ccnt[j], (0,))
                sfx = jax.lax.rev(jnp.cumsum(r), (0,)) + carry
                sgrp[j] = sfx
                carry = carry + jnp.sum(ccnt[j])
            for j in range(16):
                nb = nb + jnp.sum((sgrp[j] >= kk).astype(jnp.int32))
            bstar = nb - 1
            cnt_above = jnp.int32(0)
            sum_above = jnp.float32(0.0)
            bcnt = jnp.int32(0)
            bsum = jnp.float32(0.0)
            for j in range(16):
                binv = lanes + j * 16
                above = binv > bstar
                at_b = binv == bstar
                cnt_above = cnt_above + jnp.sum(
                    jnp.where(above, ccnt[j], 0))
                sum_above = sum_above + jnp.sum(
                    jnp.where(above, csum[j], 0.0))
                bcnt = bcnt + jnp.sum(jnp.where(at_b, ccnt[j], 0))
                bsum = bsum + jnp.sum(jnp.where(at_b, csum[j], 0.0))
            return bstar, cnt_above, sum_above, bcnt, bsum

        _zero_hists()

        @plsc.parallel_loop(0, NCHUNK, unroll=4)
        def pass1(cc):
            v = vals[pl.ds(cc * 16, 16)]
            bits = plsc.bitcast(v, jnp.int32)
            copy = (cc & 3) * 4096
            idx = jax.lax.shift_right_logical(bits, 23) + lane_base + copy
            plsc.addupdate_scatter(hcnt, [idx], ones_i)
            plsc.addupdate_scatter(hsum, [idx], v)

        bstar, cnt_ab1, sum_ab1, _, _ = _suffix_stats(k)

        _zero_hists()
        k2 = k - cnt_ab1

        @plsc.parallel_loop(0, NCHUNK, unroll=4)
        def pass2(cc):
            v = vals[pl.ds(cc * 16, 16)]
            bits = plsc.bitcast(v, jnp.int32)
            coarse = jax.lax.shift_right_logical(bits, 23)
            fine = jax.lax.shift_right_logical(bits, 15) & 0xFF
            msk = coarse == bstar
            copy = (cc & 3) * 4096
            idx = fine + lane_base + copy
            plsc.addupdate_scatter(hcnt, [idx], ones_i, mask=msk)
            plsc.addupdate_scatter(hsum, [idx], v, mask=msk)

        _, cnt_ab2, sum_ab2, bcnt, bsum = _suffix_stats(k2)

        ties_v = jnp.full((16,), (k2 - cnt_ab2).astype(jnp.float32))
        bsum_v = jnp.full((16,), bsum)
        bcnt_v = jnp.maximum(jnp.full((16,), bcnt.astype(jnp.float32)), 1.0)
        t_v = bsum_v / bcnt_v
        sum_v = jnp.full((16,), sum_ab1 + sum_ab2)
        k_v = jnp.full((16,), k)
        topk_v = jnp.where(k_v > 0, sum_v + ties_v * t_v,
                           jnp.zeros((16,), jnp.float32))
        orow[...] = topk_v
        pltpu.sync_copy(orow, out_hbm.at[img])


_sc_topk = functools.partial(
    pl.kernel,
    mesh=plsc.VectorSubcoreMesh(core_axis_name="c", subcore_axis_name="s"),
    out_type=jax.ShapeDtypeStruct((BATCH_C, 16), jnp.float32),
    scratch_types=[
        pltpu.VMEM((P_PAD,), jnp.float32),
        pltpu.VMEM((16,), jnp.float32),
        pltpu.VMEM((16384,), jnp.int32),
        pltpu.VMEM((16384,), jnp.float32),
        pltpu.VMEM((16,), jnp.float32),
        pltpu.SemaphoreType.DMA,
    ],
    compiler_params=pltpu.CompilerParams(needs_layout_passes=False),
)(_sc_topk_kernel)


@jax.jit
def kernel(odm_locs, odm_scores, attention_map, boxes, labels,
           ignored_regions, priors_cxcy):
    B, P, _ = odm_scores.shape
    pad = P_PAD - P

    locs_t = jnp.pad(jnp.transpose(odm_locs, (0, 2, 1)),
                     ((0, 0), (0, 0), (0, pad)))
    scores_t = jnp.pad(jnp.transpose(odm_scores, (0, 2, 1)),
                       ((0, 0), (0, 0), (0, pad)))
    sentinel = jnp.tile(jnp.array([[-9.0], [-9.0], [1.0], [1.0]],
                                  jnp.float32), (1, pad))
    priors_t = jnp.concatenate(
        [jnp.transpose(priors_cxcy, (1, 0)), sentinel], axis=1)
    att = attention_map.reshape(B, 1, ATT_HW)
    labels_c = labels.astype(jnp.int32).reshape(B, N_OBJ_C, 1)

    cn, npos, obj, posf, tot = pl.pallas_call(
        _image_kernel,
        grid=(B,),
        in_specs=[
            pl.BlockSpec((1, 4, P_PAD), lambda i: (i, 0, 0)),
            pl.BlockSpec((1, N_OBJ_C, 4), lambda i: (i, 0, 0)),
            pl.BlockSpec((1, N_OBJ_C, 1), lambda i: (i, 0, 0)),
            pl.BlockSpec((1, N_IGN_C, 4), lambda i: (i, 0, 0)),
            pl.BlockSpec((4, P_PAD), lambda i: (0, 0)),
        ],
        out_specs=[
            pl.BlockSpec((1, 1, P_PAD), lambda i: (i, 0, 0)),
            pl.BlockSpec((1, 1, 128), lambda i: (i, 0, 0)),
            pl.BlockSpec((1, 1, P_PAD), lambda i: (i, 0, 0)),
            pl.BlockSpec((1, 1, P_PAD), lambda i: (i, 0, 0)),
            pl.BlockSpec(memory_space=pltpu.SMEM),
        ],
        out_shape=[
            jax.ShapeDtypeStruct((B, 1, P_PAD), jnp.float32),
            jax.ShapeDtypeStruct((B, 1, 128), jnp.float32),
            jax.ShapeDtypeStruct((B, 1, P_PAD), jnp.float32),
            jax.ShapeDtypeStruct((B, 1, P_PAD), jnp.float32),
            jax.ShapeDtypeStruct((1, 2), jnp.float32),
        ],
        scratch_shapes=[pltpu.SMEM((2,), jnp.float32)],
    )(scores_t, boxes, labels_c, ignored_regions, priors_t)

    topk_rows = _sc_topk(cn.reshape(B, P_PAD), npos[:, 0, 0:16])

    tot2 = pl.pallas_call(
        _decode_kernel,
        grid=(B,),
        in_specs=[
            pl.BlockSpec((1, 4, P_PAD), lambda i: (i, 0, 0)),
            pl.BlockSpec((1, 1, ATT_HW), lambda i: (i, 0, 0)),
            pl.BlockSpec((1, N_OBJ_C, 4), lambda i: (i, 0, 0)),
            pl.BlockSpec((4, P_PAD), lambda i: (0, 0)),
            pl.BlockSpec((1, 1, P_PAD), lambda i: (i, 0, 0)),
            pl.BlockSpec((1, 1, P_PAD), lambda i: (i, 0, 0)),
        ],
        out_specs=pl.BlockSpec(memory_space=pltpu.SMEM),
        out_shape=jax.ShapeDtypeStruct((1, 2), jnp.float32),
        scratch_shapes=[pltpu.SMEM((2,), jnp.float32)],
    )(locs_t, att, boxes, priors_t, obj, posf)

    total_pos = tot[0, 0]
    loss = ((tot[0, 1] + jnp.sum(topk_rows[:, 0]) + tot2[0, 0]) / total_pos
            + tot2[0, 1])
    return loss.reshape(())
